# baseline (speedup 1.0000x reference)
import functools

import jax
import jax.numpy as jnp
from jax import lax
from jax.experimental import pallas as pl
from jax.experimental.pallas import tpu as pltpu
from jax.experimental.pallas import tpu_sc as plsc

_NUM_NODES = 100000
_D = 128
_B = 102400
_CTX = 10
_NPAIR = _CTX - 1
_EPS = 1e-15

_NW = 32
_WALKS = 2 * _B
_WPC = 16
_IDS_PER_CHUNK = _WPC * _CTX
_GRP = 80
_NGRP = _IDS_PER_CHUNK // _GRP
_CHUNKS = _WALKS // (_NW * _WPC)
_OUT_ROWS = _WPC * _NPAIR


def _sc_partial_dots(embedding, ids3d):
    mesh = plsc.VectorSubcoreMesh(core_axis_name="c", subcore_axis_name="s")

    @functools.partial(
        pl.kernel,
        out_type=jax.ShapeDtypeStruct((_WALKS * _NPAIR, 16), jnp.float32),
        mesh=mesh,
        scratch_types=[
            pltpu.VMEM((_NGRP, _GRP), jnp.int32),
            pltpu.VMEM((_NGRP, _GRP), jnp.int32),
            pltpu.VMEM((_IDS_PER_CHUNK, _D), jnp.float32),
            pltpu.VMEM((_IDS_PER_CHUNK, _D), jnp.float32),
            pltpu.VMEM((_OUT_ROWS, 16), jnp.float32),
            pltpu.VMEM((_OUT_ROWS, 16), jnp.float32),
            pltpu.SemaphoreType.DMA,
            pltpu.SemaphoreType.DMA,
            pltpu.SemaphoreType.DMA,
            pltpu.SemaphoreType.DMA,
            pltpu.SemaphoreType.DMA,
            pltpu.SemaphoreType.DMA,
        ],
    )
    def sc_kernel(table_hbm, ids_hbm, out_hbm,
                  idx0, idx1, rows0, rows1, ob0, ob1,
                  isem0, isem1, rsem0, rsem1, osem0, osem1):
        wid = lax.axis_index("s") * 2 + lax.axis_index("c")
        cc0 = wid * _CHUNKS

        idxb = (idx0, idx1)
        rowb = (rows0, rows1)
        outb = (ob0, ob1)
        isem = (isem0, isem1)
        rsem = (rsem0, rsem1)
        osem = (osem0, osem1)

        def start_gathers(b, _):
            for grp in range(_NGRP):
                pltpu.async_copy(
                    table_hbm.at[idxb[b].at[grp]],
                    rowb[b].at[pl.ds(grp * _GRP, _GRP)],
                    rsem[b],
                )

        def wait_gathers(b):
            for grp in range(_NGRP):
                pltpu.make_async_copy(
                    table_hbm.at[idxb[b].at[grp]],
                    rowb[b].at[pl.ds(grp * _GRP, _GRP)],
                    rsem[b],
                ).wait()

        def out_slice(c):
            return out_hbm.at[pl.ds((cc0 + c) * _OUT_ROWS, _OUT_ROWS)]

        def compute(b):
            rows = rowb[b]
            out = outb[b]

            @pl.loop(0, _WPC)
            def _(w):
                base = w * _CTX
                s = [rows[base, pl.ds(k * 16, 16)] for k in range(8)]
                for j in range(_NPAIR):
                    r = base + 1 + j
                    acc = s[0] * rows[r, pl.ds(0, 16)]
                    for k in range(1, 8):
                        acc = acc + s[k] * rows[r, pl.ds(k * 16, 16)]
                    out[w * _NPAIR + j, :] = acc

        def process(c, b):
            wait_gathers(b)

            @pl.when(c + 2 < _CHUNKS)
            def _(c=c, b=b):
                pltpu.async_copy(ids_hbm.at[cc0 + c + 2], idxb[b], isem[b])

            @pl.when(c >= 2)
            def _(c=c, b=b):
                pltpu.make_async_copy(outb[b], out_slice(c), osem[b]).wait()

            compute(b)
            pltpu.async_copy(outb[b], out_slice(c), osem[b])

            @pl.when(c + 2 < _CHUNKS)
            def _(c=c, b=b):
                pltpu.make_async_copy(
                    ids_hbm.at[cc0 + c + 2], idxb[b], isem[b]
                ).wait()
                start_gathers(b, None)

        pltpu.sync_copy(ids_hbm.at[cc0], idx0)
        pltpu.sync_copy(ids_hbm.at[cc0 + 1], idx1)
        start_gathers(0, None)
        start_gathers(1, None)

        @pl.loop(0, _CHUNKS, step=2)
        def _(c):
            process(c, 0)
            process(c + 1, 1)

        pltpu.make_async_copy(ob0, out_slice(_CHUNKS - 2), osem0).wait()
        pltpu.make_async_copy(ob1, out_slice(_CHUNKS - 1), osem1).wait()

    return sc_kernel(embedding, ids3d)


def _tc_loss_body(pos_ref, neg_ref, out_ref):
    lane = lax.broadcasted_iota(jnp.int32, (_D, 8), 0)
    seg = lax.broadcasted_iota(jnp.int32, (_D, 8), 1)
    fold = (lane // 16 == seg).astype(jnp.float32)

    dn = (((1,), (0,)), ((), ()))
    dots_p = lax.dot_general(pos_ref[...], fold, dn,
                             preferred_element_type=jnp.float32)
    dots_n = lax.dot_general(neg_ref[...], fold, dn,
                             preferred_element_type=jnp.float32)
    term_p = -jnp.log(jnp.maximum(jax.nn.sigmoid(dots_p), 0.0) + _EPS)
    term_n = -jnp.log(jnp.maximum(1.0 - jax.nn.sigmoid(dots_n), 0.0) + _EPS)
    part = (jnp.sum(term_p) + jnp.sum(term_n)).reshape(1, 1)

    @pl.when(pl.program_id(0) == 0)
    def _():
        out_ref[...] = jnp.zeros((1, 1), jnp.float32)

    out_ref[...] += part


def _tc_loss(partials):
    rows_total = _WALKS * _NPAIR // 8
    half = rows_total // 2
    p2 = partials.reshape(rows_total, _D)
    rb = 1152
    nblk = half // rb
    out = pl.pallas_call(
        _tc_loss_body,
        grid=(nblk,),
        in_specs=[
            pl.BlockSpec((rb, _D), lambda i: (i, 0)),
            pl.BlockSpec((rb, _D), lambda i, n=nblk: (i + n, 0)),
        ],
        out_specs=pl.BlockSpec((1, 1), lambda i: (0, 0)),
        out_shape=jax.ShapeDtypeStruct((1, 1), jnp.float32),
    )(p2, p2)
    return out[0, 0]


def kernel(pos_rw, neg_rw, embedding):
    ids = jnp.concatenate(
        [pos_rw.reshape(-1), neg_rw.reshape(-1)]
    ).astype(jnp.int32)
    ids3d = ids.reshape(_NW * _CHUNKS, _NGRP, _GRP)
    partials = _sc_partial_dots(embedding, ids3d)
    total = _tc_loss(partials)
    return total / jnp.float32(_B * _NPAIR)
